# Initial kernel scaffold; baseline (speedup 1.0000x reference)
import jax
import jax.numpy as jnp
from jax.experimental import pallas as pl


def kernel(x, edge_index, intersection_assign, history_buffer, gat_W0, gat_as0, gat_ad0, gat_b0, gat_W1, gat_as1, gat_ad1, gat_b1, t0_Wq, t0_Wk, t0_Wv, t0_Wo, t0_bq, t0_bk, t0_bv, t0_bo, t0_W1, t0_b1, t0_W2, t0_b2, t0_ln1_g, t0_ln1_b, t0_ln2_g, t0_ln2_b, t1_Wq, t1_Wk, t1_Wv, t1_Wo, t1_bq, t1_bk, t1_bv, t1_bo, t1_W1, t1_b1, t1_W2, t1_b2, t1_ln1_g, t1_ln1_b, t1_ln2_g, t1_ln2_b, actor_W1, actor_b1, actor_W2, actor_b2, critic_W1, critic_b1, critic_W2, critic_b2):
    raise NotImplementedError("write your pallas kernel here")


# speedup vs baseline: 1.0891x; 1.0891x over previous
import math

import jax
import jax.numpy as jnp
import numpy as np
from jax.experimental import pallas as pl

N_NODES = 100000
N_EDGES = 3200000
D_IN = 128
HID = 128
HEADS = 8
N_INTER = 10000
HIST = 5
TLAYERS = 2
FF = HID * 4
MAX_LEN = 100


def _pos_encoding(d_model, max_len):
    position = np.arange(max_len)[:, None].astype(np.float32)
    div_term = np.exp(np.arange(0, d_model, 2).astype(np.float32) * (-math.log(10000.0) / d_model))
    pe = np.zeros((max_len, d_model), dtype=np.float32)
    pe[:, 0::2] = np.sin(position * div_term)
    pe[:, 1::2] = np.cos(position * div_term)
    return jnp.asarray(pe)


def _mm_relu_body(x_ref, w_ref, b_ref, o_ref):
    o_ref[...] = jax.nn.relu(
        jnp.dot(x_ref[...], w_ref[...], preferred_element_type=jnp.float32)
        + b_ref[...])


def _mm_relu(x, W, b, blk=2000):
    n, k = x.shape
    m = W.shape[1]
    return pl.pallas_call(
        _mm_relu_body,
        grid=(n // blk,),
        in_specs=[
            pl.BlockSpec((blk, k), lambda i: (i, 0)),
            pl.BlockSpec((k, m), lambda i: (0, 0)),
            pl.BlockSpec((m,), lambda i: (0,)),
        ],
        out_specs=pl.BlockSpec((blk, m), lambda i: (i, 0)),
        out_shape=jax.ShapeDtypeStruct((n, m), jnp.float32),
    )(x, W, b)


def _gat_layer(x, src, dst, W, att_src, att_dst, bias):
    n, in_dim = x.shape
    W3 = W.reshape(in_dim, HEADS, HID)
    avec_src = jnp.einsum('ihd,hd->ih', W3, att_src)
    avec_dst = jnp.einsum('ihd,hd->ih', W3, att_dst)
    asrc = x @ avec_src
    adst = x @ avec_dst
    e = asrc[src] + adst[dst]
    e = jax.nn.leaky_relu(e, negative_slope=0.2)
    w = jnp.exp(e)
    denom = jax.ops.segment_sum(w, dst, num_segments=n)
    agg = jax.ops.segment_sum(w[:, :, None] * x[src][:, None, :], dst,
                              num_segments=n)
    agg = agg / jnp.maximum(denom, 1e-30)[:, :, None]
    Wp = W3.transpose(1, 0, 2).reshape(HEADS * in_dim, HID) / HEADS
    out = agg.reshape(n, HEADS * in_dim) @ Wp + bias
    return jax.nn.relu(out)


def _layer_norm(x, g, b, eps=1e-5):
    mu = x.mean(axis=-1, keepdims=True)
    var = ((x - mu) ** 2).mean(axis=-1, keepdims=True)
    return (x - mu) / jnp.sqrt(var + eps) * g + b


def _mha(x, Wq, bq, Wk, bk, Wv, bv, Wo, bo):
    B, S, D = x.shape
    dh = D // HEADS
    q = (x @ Wq + bq).reshape(B, S, HEADS, dh).transpose(0, 2, 1, 3)
    k = (x @ Wk + bk).reshape(B, S, HEADS, dh).transpose(0, 2, 1, 3)
    v = (x @ Wv + bv).reshape(B, S, HEADS, dh).transpose(0, 2, 1, 3)
    att = jax.nn.softmax(q @ k.transpose(0, 1, 3, 2) / math.sqrt(dh), axis=-1)
    o = (att @ v).transpose(0, 2, 1, 3).reshape(B, S, D)
    return o @ Wo + bo


def kernel(x, edge_index, intersection_assign, history_buffer,
           gat_W0, gat_as0, gat_ad0, gat_b0,
           gat_W1, gat_as1, gat_ad1, gat_b1,
           t0_Wq, t0_Wk, t0_Wv, t0_Wo, t0_bq, t0_bk, t0_bv, t0_bo,
           t0_W1, t0_b1, t0_W2, t0_b2, t0_ln1_g, t0_ln1_b, t0_ln2_g, t0_ln2_b,
           t1_Wq, t1_Wk, t1_Wv, t1_Wo, t1_bq, t1_bk, t1_bv, t1_bo,
           t1_W1, t1_b1, t1_W2, t1_b2, t1_ln1_g, t1_ln1_b, t1_ln2_g, t1_ln2_b,
           actor_W1, actor_b1, actor_W2, actor_b2,
           critic_W1, critic_b1, critic_W2, critic_b2):
    src, dst = edge_index[0], edge_index[1]
    h = _gat_layer(x, src, dst, gat_W0, gat_as0, gat_ad0, gat_b0)
    h = _gat_layer(h, src, dst, gat_W1, gat_as1, gat_ad1, gat_b1)
    sums = jax.ops.segment_sum(h, intersection_assign, num_segments=N_INTER)
    cnts = jax.ops.segment_sum(jnp.ones((h.shape[0],), h.dtype),
                               intersection_assign, num_segments=N_INTER)
    spatial = sums / jnp.maximum(cnts, 1.0)[:, None]
    pe = _pos_encoding(HID, MAX_LEN)
    buf = jnp.concatenate([history_buffer[1:], spatial[None]], axis=0)
    seq = (buf + pe[:HIST][:, None, :]).transpose(1, 0, 2)
    tp = dict(t0_Wq=t0_Wq, t0_Wk=t0_Wk, t0_Wv=t0_Wv, t0_Wo=t0_Wo,
              t0_bq=t0_bq, t0_bk=t0_bk, t0_bv=t0_bv, t0_bo=t0_bo,
              t0_W1=t0_W1, t0_b1=t0_b1, t0_W2=t0_W2, t0_b2=t0_b2,
              t0_ln1_g=t0_ln1_g, t0_ln1_b=t0_ln1_b, t0_ln2_g=t0_ln2_g, t0_ln2_b=t0_ln2_b,
              t1_Wq=t1_Wq, t1_Wk=t1_Wk, t1_Wv=t1_Wv, t1_Wo=t1_Wo,
              t1_bq=t1_bq, t1_bk=t1_bk, t1_bv=t1_bv, t1_bo=t1_bo,
              t1_W1=t1_W1, t1_b1=t1_b1, t1_W2=t1_W2, t1_b2=t1_b2,
              t1_ln1_g=t1_ln1_g, t1_ln1_b=t1_ln1_b, t1_ln2_g=t1_ln2_g, t1_ln2_b=t1_ln2_b)
    for l in range(TLAYERS):
        a = _mha(seq, tp[f't{l}_Wq'], tp[f't{l}_bq'], tp[f't{l}_Wk'], tp[f't{l}_bk'],
                 tp[f't{l}_Wv'], tp[f't{l}_bv'], tp[f't{l}_Wo'], tp[f't{l}_bo'])
        seq = _layer_norm(seq + a, tp[f't{l}_ln1_g'], tp[f't{l}_ln1_b'])
        ff = jax.nn.relu(seq @ tp[f't{l}_W1'] + tp[f't{l}_b1']) @ tp[f't{l}_W2'] + tp[f't{l}_b2']
        seq = _layer_norm(seq + ff, tp[f't{l}_ln2_g'], tp[f't{l}_ln2_b'])
    temporal = seq[:, -1, :]
    combined = jnp.concatenate([spatial, temporal], axis=-1)
    ah = _mm_relu(combined, actor_W1, actor_b1)
    ch = _mm_relu(combined, critic_W1, critic_b1)
    logits = ah @ actor_W2 + actor_b2
    value = ch @ critic_W2 + critic_b2
    return (jnp.squeeze(logits, -1), jnp.squeeze(value, -1))


# speedup vs baseline: 30.6119x; 28.1065x over previous
import functools
import math

import jax
import jax.numpy as jnp
import numpy as np
from jax import lax
from jax.experimental import pallas as pl
from jax.experimental.pallas import tpu as pltpu
from jax.experimental.pallas import tpu_sc as plsc

N_NODES = 100000
N_EDGES = 3200000
D_IN = 128
HID = 128
HEADS = 8
N_INTER = 10000
HIST = 5
TLAYERS = 2
FF = HID * 4
MAX_LEN = 100

NW = 32
DPT = N_NODES // NW
CHUNK = 128
EPAD = N_EDGES + 2 * CHUNK


def _pos_encoding(d_model, max_len):
    position = np.arange(max_len)[:, None].astype(np.float32)
    div_term = np.exp(np.arange(0, d_model, 2).astype(np.float32) * (-math.log(10000.0) / d_model))
    pe = np.zeros((max_len, d_model), dtype=np.float32)
    pe[:, 0::2] = np.sin(position * div_term)
    pe[:, 1::2] = np.cos(position * div_term)
    return jnp.asarray(pe)


def _mm_relu_body(x_ref, w_ref, b_ref, o_ref):
    o_ref[...] = jax.nn.relu(
        jnp.dot(x_ref[...], w_ref[...], preferred_element_type=jnp.float32)
        + b_ref[...])


def _mm_relu(x, W, b, blk=1000):
    n, k = x.shape
    m = W.shape[1]
    return pl.pallas_call(
        _mm_relu_body,
        grid=(n // blk,),
        in_specs=[
            pl.BlockSpec((blk, k), lambda i: (i, 0)),
            pl.BlockSpec((k, m), lambda i: (0, 0)),
            pl.BlockSpec((m,), lambda i: (0,)),
        ],
        out_specs=pl.BlockSpec((blk, m), lambda i: (i, 0)),
        out_shape=jax.ShapeDtypeStruct((n, m), jnp.float32),
    )(x, W, b)


def _attn_vec_body(h_ref, w_ref, as_ref, ad_ref, s_ref, d_ref):
    W = w_ref[...]
    in_dim = W.shape[0]
    avs = (W * as_ref[...].reshape(1, HEADS * HID)).reshape(in_dim, HEADS, HID).sum(axis=2)
    avd = (W * ad_ref[...].reshape(1, HEADS * HID)).reshape(in_dim, HEADS, HID).sum(axis=2)
    hv = h_ref[...]
    z = jnp.zeros((hv.shape[0], 8), jnp.float32)
    s_ref[...] = jnp.concatenate([jnp.dot(hv, avs, preferred_element_type=jnp.float32), z], axis=1)
    d_ref[...] = jnp.concatenate([jnp.dot(hv, avd, preferred_element_type=jnp.float32), z], axis=1)


def _attn_vecs(h, W, att_src, att_dst, blk=2000):
    n, in_dim = h.shape
    return pl.pallas_call(
        _attn_vec_body,
        grid=(n // blk,),
        in_specs=[
            pl.BlockSpec((blk, in_dim), lambda i: (i, 0)),
            pl.BlockSpec((in_dim, HEADS * HID), lambda i: (0, 0)),
            pl.BlockSpec((HEADS, HID), lambda i: (0, 0)),
            pl.BlockSpec((HEADS, HID), lambda i: (0, 0)),
        ],
        out_specs=[pl.BlockSpec((blk, 16), lambda i: (i, 0)),
                   pl.BlockSpec((blk, 16), lambda i: (i, 0))],
        out_shape=[jax.ShapeDtypeStruct((n, 16), jnp.float32),
                   jax.ShapeDtypeStruct((n, 16), jnp.float32)],
    )(h, W, att_src, att_dst)


def _edge_body(x_hbm, as_hbm, ad_hbm, src_hbm, dst_hbm, bnd_hbm, agg_hbm,
               idx_v, dst_v, xg_v, ag_v, adl_v, acc, dw, obuf, zbuf, bnd_v,
               sem1, sem2):
    wid = lax.axis_index("s") * 2 + lax.axis_index("c")
    d_base = wid * DPT
    d_end = d_base + DPT
    z16 = jnp.zeros((16,), jnp.float32)

    pltpu.sync_copy(bnd_hbm, bnd_v)
    bvec = bnd_v[pl.ds(wid, 16)]
    start = bvec[0]
    end = bvec[1]
    start0 = (start // 8) * 8
    nchunks = (end - start0 + CHUNK - 1) // CHUNK

    pltpu.sync_copy(ad_hbm.at[pl.ds(d_base, DPT)], adl_v)
    for j in range(64):
        zbuf[pl.ds(16 * j, 16)] = z16
        acc[j // 8, pl.ds(16 * (j % 8), 16)] = z16
    dw[pl.ds(0, 16)] = z16

    def _flush(prev, d):
        @pl.when(prev >= d_base)
        def _():
            rd = 1.0 / jnp.maximum(dw[pl.ds(0, 16)], 1e-16)
            for h in range(8):
                r = rd[h]
                for j in range(8):
                    obuf[pl.ds(h * 128 + 16 * j, 16)] = acc[h, pl.ds(16 * j, 16)] * r
                    acc[h, pl.ds(16 * j, 16)] = z16
            dw[pl.ds(0, 16)] = z16
            pltpu.sync_copy(obuf, agg_hbm.at[prev])

        def _gap(g, c):
            pltpu.sync_copy(zbuf, agg_hbm.at[g])
            return c
        lax.fori_loop(jnp.maximum(prev + 1, d_base), d, _gap, 0)

    def _chunk(c, prev):
        off = start0 + c * CHUNK
        pltpu.sync_copy(src_hbm.at[pl.ds(off, CHUNK)], idx_v)
        pltpu.sync_copy(dst_hbm.at[pl.ds(off, CHUNK)], dst_v.at[pl.ds(0, CHUNK)])
        cp1 = pltpu.async_copy(x_hbm.at[idx_v], xg_v, sem1)
        cp2 = pltpu.async_copy(as_hbm.at[idx_v], ag_v, sem2)
        cp1.wait()
        cp2.wait()
        cnt = jnp.minimum(CHUNK, end - off)

        def _edge(e, prev):
            d = dst_v[pl.ds(e, 16)][0]

            def _proc(prev):
                prev = lax.cond(d != prev,
                                lambda: (_flush(prev, d), d)[1],
                                lambda: prev)
                attn = ag_v[e, :] + adl_v[d - d_base, :]
                attn = jnp.where(attn > 0, attn, attn * 0.2)
                w = jnp.exp(attn)
                plsc.addupdate(dw.at[pl.ds(0, 16)], w)
                xj = [xg_v[e, pl.ds(16 * j, 16)] for j in range(8)]
                for h in range(8):
                    ws = w[h]
                    for j in range(8):
                        plsc.addupdate(acc.at[h, pl.ds(16 * j, 16)], ws * xj[j])
                return prev

            return lax.cond(d >= d_base, _proc, lambda p: p, prev)

        return lax.fori_loop(0, cnt, _edge, prev)

    prev = lax.fori_loop(0, nchunks, _chunk, d_base - 1)
    _flush(prev, d_end)


def _gat_edge_sc(x, asrc, adst, srcp, dstp, bounds):
    mesh = plsc.VectorSubcoreMesh(core_axis_name="c", subcore_axis_name="s")
    f = pl.kernel(
        _edge_body,
        out_type=jax.ShapeDtypeStruct((N_NODES, HEADS * D_IN), jnp.float32),
        mesh=mesh,
        compiler_params=pltpu.CompilerParams(use_tc_tiling_on_sc=False),
        scratch_types=[
            pltpu.VMEM((CHUNK,), jnp.int32),
            pltpu.VMEM((CHUNK + 16,), jnp.int32),
            pltpu.VMEM((CHUNK, 128), jnp.float32),
            pltpu.VMEM((CHUNK, 16), jnp.float32),
            pltpu.VMEM((DPT, 16), jnp.float32),
            pltpu.VMEM((8, 128), jnp.float32),
            pltpu.VMEM((48,), jnp.float32),
            pltpu.VMEM((1024,), jnp.float32),
            pltpu.VMEM((1024,), jnp.float32),
            pltpu.VMEM((64,), jnp.int32),
            pltpu.SemaphoreType.DMA,
            pltpu.SemaphoreType.DMA,
        ],
    )
    return f(x, asrc, adst, srcp, dstp, bounds)


def _gat_layer(x, srcp, dstp, bounds, W, att_src, att_dst, bias):
    asrc, adst = _attn_vecs(x, W, att_src, att_dst)
    agg = _gat_edge_sc(x, asrc, adst, srcp, dstp, bounds)
    in_dim = x.shape[1]
    W3 = W.reshape(in_dim, HEADS, HID)
    Wp = W3.transpose(1, 0, 2).reshape(HEADS * in_dim, HID) / HEADS
    return _mm_relu(agg, Wp, bias)


def _layer_norm(x, g, b, eps=1e-5):
    mu = x.mean(axis=-1, keepdims=True)
    var = ((x - mu) ** 2).mean(axis=-1, keepdims=True)
    return (x - mu) / jnp.sqrt(var + eps) * g + b


def _mha(x, Wq, bq, Wk, bk, Wv, bv, Wo, bo):
    B, S, D = x.shape
    dh = D // HEADS
    q = (x @ Wq + bq).reshape(B, S, HEADS, dh).transpose(0, 2, 1, 3)
    k = (x @ Wk + bk).reshape(B, S, HEADS, dh).transpose(0, 2, 1, 3)
    v = (x @ Wv + bv).reshape(B, S, HEADS, dh).transpose(0, 2, 1, 3)
    att = jax.nn.softmax(q @ k.transpose(0, 1, 3, 2) / math.sqrt(dh), axis=-1)
    o = (att @ v).transpose(0, 2, 1, 3).reshape(B, S, D)
    return o @ Wo + bo


def kernel(x, edge_index, intersection_assign, history_buffer,
           gat_W0, gat_as0, gat_ad0, gat_b0,
           gat_W1, gat_as1, gat_ad1, gat_b1,
           t0_Wq, t0_Wk, t0_Wv, t0_Wo, t0_bq, t0_bk, t0_bv, t0_bo,
           t0_W1, t0_b1, t0_W2, t0_b2, t0_ln1_g, t0_ln1_b, t0_ln2_g, t0_ln2_b,
           t1_Wq, t1_Wk, t1_Wv, t1_Wo, t1_bq, t1_bk, t1_bv, t1_bo,
           t1_W1, t1_b1, t1_W2, t1_b2, t1_ln1_g, t1_ln1_b, t1_ln2_g, t1_ln2_b,
           actor_W1, actor_b1, actor_W2, actor_b2,
           critic_W1, critic_b1, critic_W2, critic_b2):
    src = edge_index[0].astype(jnp.int32)
    dst = edge_index[1].astype(jnp.int32)
    order = jnp.argsort(dst)
    dstS = dst[order]
    srcS = src[order]
    srcp = jnp.zeros((EPAD,), jnp.int32).at[:N_EDGES].set(srcS)
    dstp = jnp.full((EPAD,), N_NODES, jnp.int32).at[:N_EDGES].set(dstS)
    bounds = jnp.zeros((64,), jnp.int32).at[:NW + 1].set(
        jnp.searchsorted(dstS, (jnp.arange(NW + 1) * DPT).astype(jnp.int32)).astype(jnp.int32))

    h = _gat_layer(x, srcp, dstp, bounds, gat_W0, gat_as0, gat_ad0, gat_b0)
    h = _gat_layer(h, srcp, dstp, bounds, gat_W1, gat_as1, gat_ad1, gat_b1)

    sums = jax.ops.segment_sum(h, intersection_assign, num_segments=N_INTER)
    cnts = jax.ops.segment_sum(jnp.ones((h.shape[0],), h.dtype),
                               intersection_assign, num_segments=N_INTER)
    spatial = sums / jnp.maximum(cnts, 1.0)[:, None]

    pe = _pos_encoding(HID, MAX_LEN)
    buf = jnp.concatenate([history_buffer[1:], spatial[None]], axis=0)
    seq = (buf + pe[:HIST][:, None, :]).transpose(1, 0, 2)
    tp = dict(t0_Wq=t0_Wq, t0_Wk=t0_Wk, t0_Wv=t0_Wv, t0_Wo=t0_Wo,
              t0_bq=t0_bq, t0_bk=t0_bk, t0_bv=t0_bv, t0_bo=t0_bo,
              t0_W1=t0_W1, t0_b1=t0_b1, t0_W2=t0_W2, t0_b2=t0_b2,
              t0_ln1_g=t0_ln1_g, t0_ln1_b=t0_ln1_b, t0_ln2_g=t0_ln2_g, t0_ln2_b=t0_ln2_b,
              t1_Wq=t1_Wq, t1_Wk=t1_Wk, t1_Wv=t1_Wv, t1_Wo=t1_Wo,
              t1_bq=t1_bq, t1_bk=t1_bk, t1_bv=t1_bv, t1_bo=t1_bo,
              t1_W1=t1_W1, t1_b1=t1_b1, t1_W2=t1_W2, t1_b2=t1_b2,
              t1_ln1_g=t1_ln1_g, t1_ln1_b=t1_ln1_b, t1_ln2_g=t1_ln2_g, t1_ln2_b=t1_ln2_b)
    for l in range(TLAYERS):
        a = _mha(seq, tp[f't{l}_Wq'], tp[f't{l}_bq'], tp[f't{l}_Wk'], tp[f't{l}_bk'],
                 tp[f't{l}_Wv'], tp[f't{l}_bv'], tp[f't{l}_Wo'], tp[f't{l}_bo'])
        seq = _layer_norm(seq + a, tp[f't{l}_ln1_g'], tp[f't{l}_ln1_b'])
        ff = jax.nn.relu(seq @ tp[f't{l}_W1'] + tp[f't{l}_b1']) @ tp[f't{l}_W2'] + tp[f't{l}_b2']
        seq = _layer_norm(seq + ff, tp[f't{l}_ln2_g'], tp[f't{l}_ln2_b'])
    temporal = seq[:, -1, :]
    combined = jnp.concatenate([spatial, temporal], axis=-1)
    ah = _mm_relu(combined, actor_W1, actor_b1)
    ch = _mm_relu(combined, critic_W1, critic_b1)
    logits = ah @ actor_W2 + actor_b2
    value = ch @ critic_W2 + critic_b2
    return (jnp.squeeze(logits, -1), jnp.squeeze(value, -1))


# speedup vs baseline: 31.2726x; 1.0216x over previous
import functools
import math

import jax
import jax.numpy as jnp
import numpy as np
from jax import lax
from jax.experimental import pallas as pl
from jax.experimental.pallas import tpu as pltpu
from jax.experimental.pallas import tpu_sc as plsc

N_NODES = 100000
N_EDGES = 3200000
D_IN = 128
HID = 128
HEADS = 8
N_INTER = 10000
HIST = 5
TLAYERS = 2
FF = HID * 4
MAX_LEN = 100

NW = 32
DPT = N_NODES // NW
CHUNK = 128
EPAD = N_EDGES + 2 * CHUNK


def _pos_encoding(d_model, max_len):
    position = np.arange(max_len)[:, None].astype(np.float32)
    div_term = np.exp(np.arange(0, d_model, 2).astype(np.float32) * (-math.log(10000.0) / d_model))
    pe = np.zeros((max_len, d_model), dtype=np.float32)
    pe[:, 0::2] = np.sin(position * div_term)
    pe[:, 1::2] = np.cos(position * div_term)
    return jnp.asarray(pe)


def _mm_relu_body(x_ref, w_ref, b_ref, o_ref):
    o_ref[...] = jax.nn.relu(
        jnp.dot(x_ref[...], w_ref[...], preferred_element_type=jnp.float32)
        + b_ref[...])


def _mm_relu(x, W, b, blk=1000):
    n, k = x.shape
    m = W.shape[1]
    return pl.pallas_call(
        _mm_relu_body,
        grid=(n // blk,),
        in_specs=[
            pl.BlockSpec((blk, k), lambda i: (i, 0)),
            pl.BlockSpec((k, m), lambda i: (0, 0)),
            pl.BlockSpec((m,), lambda i: (0,)),
        ],
        out_specs=pl.BlockSpec((blk, m), lambda i: (i, 0)),
        out_shape=jax.ShapeDtypeStruct((n, m), jnp.float32),
    )(x, W, b)


def _attn_vec_body(h_ref, w_ref, as_ref, ad_ref, s_ref, d_ref):
    W = w_ref[...]
    in_dim = W.shape[0]
    avs = (W * as_ref[...].reshape(1, HEADS * HID)).reshape(in_dim, HEADS, HID).sum(axis=2)
    avd = (W * ad_ref[...].reshape(1, HEADS * HID)).reshape(in_dim, HEADS, HID).sum(axis=2)
    hv = h_ref[...]
    z = jnp.zeros((hv.shape[0], 8), jnp.float32)
    s_ref[...] = jnp.concatenate([jnp.dot(hv, avs, preferred_element_type=jnp.float32), z], axis=1)
    d_ref[...] = jnp.concatenate([jnp.dot(hv, avd, preferred_element_type=jnp.float32), z], axis=1)


def _attn_vecs(h, W, att_src, att_dst, blk=2000):
    n, in_dim = h.shape
    return pl.pallas_call(
        _attn_vec_body,
        grid=(n // blk,),
        in_specs=[
            pl.BlockSpec((blk, in_dim), lambda i: (i, 0)),
            pl.BlockSpec((in_dim, HEADS * HID), lambda i: (0, 0)),
            pl.BlockSpec((HEADS, HID), lambda i: (0, 0)),
            pl.BlockSpec((HEADS, HID), lambda i: (0, 0)),
        ],
        out_specs=[pl.BlockSpec((blk, 16), lambda i: (i, 0)),
                   pl.BlockSpec((blk, 16), lambda i: (i, 0))],
        out_shape=[jax.ShapeDtypeStruct((n, 16), jnp.float32),
                   jax.ShapeDtypeStruct((n, 16), jnp.float32)],
    )(h, W, att_src, att_dst)


WROWS = 25


def _edge_body(x_hbm, as_hbm, ad_hbm, src_hbm, dst_hbm, bnd_hbm, agg_hbm,
               idx0, idx1, dst0, dst1, xg0, xg1, ag0, ag1,
               adl_v, acc, dw, outblk, bnd_v,
               s1a, s2a, s1b, s2b):
    wid = lax.axis_index("s") * 2 + lax.axis_index("c")
    d_base = wid * DPT
    d_end = d_base + DPT
    z16 = jnp.zeros((16,), jnp.float32)
    idx = [idx0, idx1]
    dstb = [dst0, dst1]
    xg = [xg0, xg1]
    ag = [ag0, ag1]
    sems = [(s1a, s2a), (s1b, s2b)]

    pltpu.sync_copy(bnd_hbm, bnd_v)
    bvec = bnd_v[pl.ds(wid, 16)]
    start = bvec[0]
    end = bvec[1]
    start0 = (start // 8) * 8
    nchunks = (end - start0 + CHUNK - 1) // CHUNK

    pltpu.sync_copy(ad_hbm.at[pl.ds(d_base, DPT)], adl_v)
    for j in range(64):
        acc[j // 8, pl.ds(16 * (j % 8), 16)] = z16
    dw[pl.ds(0, 16)] = z16

    def _zero_outblk_row(r, c):
        for j in range(64):
            outblk[r, pl.ds(16 * j, 16)] = z16
        return c
    lax.fori_loop(0, WROWS, _zero_outblk_row, 0)

    def _flush(prev, d, wb):
        @pl.when(prev >= d_base)
        def _():
            rd = 1.0 / jnp.maximum(dw[pl.ds(0, 16)], 1e-16)
            row = prev - d_base - wb
            for h in range(8):
                r = rd[h]
                for j in range(8):
                    outblk[row, pl.ds(h * 128 + 16 * j, 16)] = acc[h, pl.ds(16 * j, 16)] * r
                    acc[h, pl.ds(16 * j, 16)] = z16
            dw[pl.ds(0, 16)] = z16

        reld = jnp.minimum(d - d_base, DPT)

        def _emit(i, wb):
            pltpu.sync_copy(outblk, agg_hbm.at[pl.ds(d_base + wb, WROWS)])
            lax.fori_loop(0, WROWS, _zero_outblk_row, 0)
            return wb + WROWS
        return lax.fori_loop(0, (reld - wb) // WROWS, _emit, wb)

    def _issue(c, b):
        off = start0 + c * CHUNK
        pltpu.sync_copy(src_hbm.at[pl.ds(off, CHUNK)], idx[b])
        pltpu.sync_copy(dst_hbm.at[pl.ds(off, CHUNK)], dstb[b].at[pl.ds(0, CHUNK)])
        pltpu.async_copy(x_hbm.at[idx[b]], xg[b], sems[b][0])
        pltpu.async_copy(as_hbm.at[idx[b]], ag[b], sems[b][1])

    def _wait(b):
        pltpu.make_async_copy(x_hbm.at[idx[b]], xg[b], sems[b][0]).wait()
        pltpu.make_async_copy(as_hbm.at[idx[b]], ag[b], sems[b][1]).wait()

    def _process(c, b, carry):
        cnt = jnp.minimum(CHUNK, end - (start0 + c * CHUNK))
        dv, xv, av = dstb[b], xg[b], ag[b]

        def _edge(e, carry):
            prev, wb = carry
            d = dv[pl.ds(e, 16)][0]

            def _proc(prev, wb):
                prev, wb = lax.cond(
                    d != prev,
                    lambda: (d, _flush(prev, d, wb)),
                    lambda: (prev, wb))
                attn = av[e, :] + adl_v[d - d_base, :]
                attn = jnp.where(attn > 0, attn, attn * 0.2)
                w = jnp.exp(attn)
                plsc.addupdate(dw.at[pl.ds(0, 16)], w)
                xj = [xv[e, pl.ds(16 * j, 16)] for j in range(8)]
                for h in range(8):
                    ws = w[h]
                    for j in range(8):
                        plsc.addupdate(acc.at[h, pl.ds(16 * j, 16)], ws * xj[j])
                return prev, wb

            return lax.cond(d >= d_base, _proc, lambda p, w: (p, w), prev, wb)

        return lax.fori_loop(0, cnt, _edge, carry)

    @pl.when(nchunks > 0)
    def _():
        _issue(0, 0)

    def _pair(k, carry):
        c0 = 2 * k

        @pl.when(c0 + 1 < nchunks)
        def _():
            _issue(c0 + 1, 1)
        _wait(0)
        carry = _process(c0, 0, carry)

        @pl.when(c0 + 2 < nchunks)
        def _():
            _issue(c0 + 2, 0)

        def _second(carry):
            _wait(1)
            return _process(c0 + 1, 1, carry)
        return lax.cond(c0 + 1 < nchunks, _second, lambda c: c, carry)

    carry = lax.fori_loop(0, (nchunks + 1) // 2, _pair, (d_base - 1, 0))
    prev, wb = carry
    _flush(prev, d_end, wb)


def _gat_edge_sc(x, asrc, adst, srcp, dstp, bounds):
    mesh = plsc.VectorSubcoreMesh(core_axis_name="c", subcore_axis_name="s")
    f = pl.kernel(
        _edge_body,
        out_type=jax.ShapeDtypeStruct((N_NODES, HEADS * D_IN), jnp.float32),
        mesh=mesh,
        compiler_params=pltpu.CompilerParams(use_tc_tiling_on_sc=False),
        scratch_types=[
            pltpu.VMEM((CHUNK,), jnp.int32),
            pltpu.VMEM((CHUNK,), jnp.int32),
            pltpu.VMEM((CHUNK + 16,), jnp.int32),
            pltpu.VMEM((CHUNK + 16,), jnp.int32),
            pltpu.VMEM((CHUNK, 128), jnp.float32),
            pltpu.VMEM((CHUNK, 128), jnp.float32),
            pltpu.VMEM((CHUNK, 16), jnp.float32),
            pltpu.VMEM((CHUNK, 16), jnp.float32),
            pltpu.VMEM((DPT, 16), jnp.float32),
            pltpu.VMEM((8, 128), jnp.float32),
            pltpu.VMEM((48,), jnp.float32),
            pltpu.VMEM((WROWS, 1024), jnp.float32),
            pltpu.VMEM((64,), jnp.int32),
            pltpu.SemaphoreType.DMA,
            pltpu.SemaphoreType.DMA,
            pltpu.SemaphoreType.DMA,
            pltpu.SemaphoreType.DMA,
        ],
    )
    return f(x, asrc, adst, srcp, dstp, bounds)


def _gat_layer(x, srcp, dstp, bounds, W, att_src, att_dst, bias):
    asrc, adst = _attn_vecs(x, W, att_src, att_dst)
    agg = _gat_edge_sc(x, asrc, adst, srcp, dstp, bounds)
    in_dim = x.shape[1]
    W3 = W.reshape(in_dim, HEADS, HID)
    Wp = W3.transpose(1, 0, 2).reshape(HEADS * in_dim, HID) / HEADS
    return _mm_relu(agg, Wp, bias)


def _layer_norm(x, g, b, eps=1e-5):
    mu = x.mean(axis=-1, keepdims=True)
    var = ((x - mu) ** 2).mean(axis=-1, keepdims=True)
    return (x - mu) / jnp.sqrt(var + eps) * g + b


def _mha(x, Wq, bq, Wk, bk, Wv, bv, Wo, bo):
    B, S, D = x.shape
    dh = D // HEADS
    q = (x @ Wq + bq).reshape(B, S, HEADS, dh).transpose(0, 2, 1, 3)
    k = (x @ Wk + bk).reshape(B, S, HEADS, dh).transpose(0, 2, 1, 3)
    v = (x @ Wv + bv).reshape(B, S, HEADS, dh).transpose(0, 2, 1, 3)
    att = jax.nn.softmax(q @ k.transpose(0, 1, 3, 2) / math.sqrt(dh), axis=-1)
    o = (att @ v).transpose(0, 2, 1, 3).reshape(B, S, D)
    return o @ Wo + bo


def kernel(x, edge_index, intersection_assign, history_buffer,
           gat_W0, gat_as0, gat_ad0, gat_b0,
           gat_W1, gat_as1, gat_ad1, gat_b1,
           t0_Wq, t0_Wk, t0_Wv, t0_Wo, t0_bq, t0_bk, t0_bv, t0_bo,
           t0_W1, t0_b1, t0_W2, t0_b2, t0_ln1_g, t0_ln1_b, t0_ln2_g, t0_ln2_b,
           t1_Wq, t1_Wk, t1_Wv, t1_Wo, t1_bq, t1_bk, t1_bv, t1_bo,
           t1_W1, t1_b1, t1_W2, t1_b2, t1_ln1_g, t1_ln1_b, t1_ln2_g, t1_ln2_b,
           actor_W1, actor_b1, actor_W2, actor_b2,
           critic_W1, critic_b1, critic_W2, critic_b2):
    src = edge_index[0].astype(jnp.int32)
    dst = edge_index[1].astype(jnp.int32)
    order = jnp.argsort(dst)
    dstS = dst[order]
    srcS = src[order]
    srcp = jnp.zeros((EPAD,), jnp.int32).at[:N_EDGES].set(srcS)
    dstp = jnp.full((EPAD,), N_NODES, jnp.int32).at[:N_EDGES].set(dstS)
    bounds = jnp.zeros((64,), jnp.int32).at[:NW + 1].set(
        jnp.searchsorted(dstS, (jnp.arange(NW + 1) * DPT).astype(jnp.int32)).astype(jnp.int32))

    h = _gat_layer(x, srcp, dstp, bounds, gat_W0, gat_as0, gat_ad0, gat_b0)
    h = _gat_layer(h, srcp, dstp, bounds, gat_W1, gat_as1, gat_ad1, gat_b1)

    sums = jax.ops.segment_sum(h, intersection_assign, num_segments=N_INTER)
    cnts = jax.ops.segment_sum(jnp.ones((h.shape[0],), h.dtype),
                               intersection_assign, num_segments=N_INTER)
    spatial = sums / jnp.maximum(cnts, 1.0)[:, None]

    pe = _pos_encoding(HID, MAX_LEN)
    buf = jnp.concatenate([history_buffer[1:], spatial[None]], axis=0)
    seq = (buf + pe[:HIST][:, None, :]).transpose(1, 0, 2)
    tp = dict(t0_Wq=t0_Wq, t0_Wk=t0_Wk, t0_Wv=t0_Wv, t0_Wo=t0_Wo,
              t0_bq=t0_bq, t0_bk=t0_bk, t0_bv=t0_bv, t0_bo=t0_bo,
              t0_W1=t0_W1, t0_b1=t0_b1, t0_W2=t0_W2, t0_b2=t0_b2,
              t0_ln1_g=t0_ln1_g, t0_ln1_b=t0_ln1_b, t0_ln2_g=t0_ln2_g, t0_ln2_b=t0_ln2_b,
              t1_Wq=t1_Wq, t1_Wk=t1_Wk, t1_Wv=t1_Wv, t1_Wo=t1_Wo,
              t1_bq=t1_bq, t1_bk=t1_bk, t1_bv=t1_bv, t1_bo=t1_bo,
              t1_W1=t1_W1, t1_b1=t1_b1, t1_W2=t1_W2, t1_b2=t1_b2,
              t1_ln1_g=t1_ln1_g, t1_ln1_b=t1_ln1_b, t1_ln2_g=t1_ln2_g, t1_ln2_b=t1_ln2_b)
    for l in range(TLAYERS):
        a = _mha(seq, tp[f't{l}_Wq'], tp[f't{l}_bq'], tp[f't{l}_Wk'], tp[f't{l}_bk'],
                 tp[f't{l}_Wv'], tp[f't{l}_bv'], tp[f't{l}_Wo'], tp[f't{l}_bo'])
        seq = _layer_norm(seq + a, tp[f't{l}_ln1_g'], tp[f't{l}_ln1_b'])
        ff = jax.nn.relu(seq @ tp[f't{l}_W1'] + tp[f't{l}_b1']) @ tp[f't{l}_W2'] + tp[f't{l}_b2']
        seq = _layer_norm(seq + ff, tp[f't{l}_ln2_g'], tp[f't{l}_ln2_b'])
    temporal = seq[:, -1, :]
    combined = jnp.concatenate([spatial, temporal], axis=-1)
    ah = _mm_relu(combined, actor_W1, actor_b1)
    ch = _mm_relu(combined, critic_W1, critic_b1)
    logits = ah @ actor_W2 + actor_b2
    value = ch @ critic_W2 + critic_b2
    return (jnp.squeeze(logits, -1), jnp.squeeze(value, -1))
